# baseline (speedup 1.0000x reference)
import functools

import jax
import jax.numpy as jnp
import numpy as np
from jax import lax
from jax.experimental import pallas as pl
from jax.experimental.pallas import tpu as pltpu
from jax.experimental.pallas import tpu_sc as plsc

NUM_BASIS = 8
R_MAX = 6.0
NC, NS, L = 2, 16, 16
NW = NC * NS


def _f(v):
    return jnp.float32(v)


def _make_run(n_nodes, n_edges, num_types, embed_dim, c, interpret=False):
    e_tile = n_edges // NW
    nchunk = e_tile // c
    assert e_tile * NW == n_edges and nchunk * c == e_tile and c % L == 0

    def _sc_body(pos_hbm, ei_hbm, te_hbm, w_hbm, out_hbm,
                 te0_v, te1_v, w_sh, pos_sh, w_s, src_v, dst_v, psrc_v,
                 pdst_v, out_v, sem1, sem2):
        cid = lax.axis_index("c")
        sid = lax.axis_index("s")
        wid = sid * NC + cid
        base0 = wid * e_tile

        pltpu.sync_copy(te_hbm.at[0], te0_v)
        pltpu.sync_copy(te_hbm.at[1], te1_v)

        @pl.when(sid == 0)
        def _stage_shared():
            pltpu.sync_copy(w_hbm, w_sh)
            pltpu.sync_copy(pos_hbm, pos_sh)

        plsc.subcore_barrier()
        pltpu.sync_copy(w_sh, w_s)

        iota = lax.iota(jnp.int32, L)
        col = [jnp.full((L,), j, jnp.int32) for j in range(4)]

        nsub = c // 80

        @pl.loop(0, nchunk)
        def _chunk(i):
            base = base0 + i * c
            di1 = pltpu.async_copy(ei_hbm.at[pl.ds(base, c)], src_v, sem1)
            di2 = pltpu.async_copy(ei_hbm.at[pl.ds(n_edges + base, c)],
                                   dst_v, sem1)
            di1.wait()
            di2.wait()
            descs = []
            for j in range(nsub):
                sl = pl.ds(j * 80, 80)
                descs.append(pltpu.async_copy(
                    pos_hbm.at[src_v.at[sl]], psrc_v.at[sl, :], sem2))
                descs.append(pltpu.async_copy(
                    pos_hbm.at[dst_v.at[sl]], pdst_v.at[sl, :], sem2))
            for d in descs:
                d.wait()

            for g in range(c // L):
                rows = iota + g * L
                x1 = plsc.load_gather(psrc_v, [rows, col[0]])
                y1 = plsc.load_gather(psrc_v, [rows, col[1]])
                z1 = plsc.load_gather(psrc_v, [rows, col[2]])
                x2 = plsc.load_gather(pdst_v, [rows, col[0]])
                y2 = plsc.load_gather(pdst_v, [rows, col[1]])
                z2 = plsc.load_gather(pdst_v, [rows, col[2]])
                tsrc = plsc.load_gather(psrc_v, [rows, col[3]]).astype(
                    jnp.int32)
                tdst = plsc.load_gather(pdst_v, [rows, col[3]]).astype(
                    jnp.int32)

                dx = x2 - x1
                dy = y2 - y1
                dz = z2 - z1
                d2 = dx * dx + dy * dy + dz * dz + _f(1e-12)

                ii = plsc.bitcast(d2, jnp.int32)
                ii = jnp.int32(0x5F3759DF) - lax.shift_right_logical(ii, 1)
                y = plsc.bitcast(ii, jnp.float32)
                for _ in range(4):
                    y = y * (_f(1.5) - _f(0.5) * d2 * y * y)
                x = d2 * y

                theta = jnp.minimum(x, _f(R_MAX)) * _f(np.pi / R_MAX)
                r = jnp.minimum(theta, _f(np.pi) - theta)
                p2 = r * r
                s = _f(1.0 / 362880.0)
                s = s * p2 + _f(-1.0 / 5040.0)
                s = s * p2 + _f(1.0 / 120.0)
                s = s * p2 + _f(-1.0 / 6.0)
                sin_t = r + r * p2 * s
                cpoly = _f(-1.0 / 3628800.0)
                cpoly = cpoly * p2 + _f(1.0 / 40320.0)
                cpoly = cpoly * p2 + _f(-1.0 / 720.0)
                cpoly = cpoly * p2 + _f(1.0 / 24.0)
                cpoly = cpoly * p2 + _f(-0.5)
                cos_r = _f(1.0) + p2 * cpoly
                twoc = jnp.where(theta < _f(np.pi / 2.0), cos_r + cos_r,
                                 -(cos_r + cos_r))

                sins = [sin_t, twoc * sin_t]
                for _ in range(NUM_BASIS - 2):
                    sins.append(twoc * sins[-1] - sins[-2])

                xs = x * _f(1.0 / R_MAX)
                xs2 = xs * xs
                xs4 = xs2 * xs2
                xs6 = xs4 * xs2
                xs7 = xs6 * xs
                xs8 = xs7 * xs
                cut = (_f(1.0) - _f(28.0) * xs6 + _f(48.0) * xs7
                       - _f(21.0) * xs8)
                cut = jnp.where(xs < _f(1.0), cut, _f(0.0))

                m_all = cut * (y * _f(2.0 / R_MAX))

                for j in range(embed_dim):
                    acc = sins[0] * w_s[0, j]
                    for k in range(1, NUM_BASIS):
                        acc = acc + sins[k] * w_s[k, j]
                    if j < embed_dim // 2:
                        tej = plsc.load_gather(
                            te0_v, [tsrc, jnp.full((L,), j, jnp.int32)])
                    else:
                        tej = plsc.load_gather(
                            te1_v,
                            [tdst, jnp.full((L,), j - embed_dim // 2,
                                            jnp.int32)])
                    o = acc * m_all * tej
                    plsc.store_scatter(
                        out_v, [rows, jnp.full((L,), j, jnp.int32)], o)

            pltpu.sync_copy(out_v, out_hbm.at[pl.ds(base, c), :])

    mesh = plsc.VectorSubcoreMesh(core_axis_name="c", subcore_axis_name="s",
                                  num_cores=NC, num_subcores=NS)
    half = embed_dim // 2
    return pl.kernel(
        _sc_body,
        out_type=jax.ShapeDtypeStruct((n_edges, embed_dim), jnp.float32),
        mesh=mesh,
        interpret=interpret,
        compiler_params=pltpu.CompilerParams(needs_layout_passes=False,
                                             use_tc_tiling_on_sc=False),
        scratch_types=[
            pltpu.VMEM((num_types, half), jnp.float32),
            pltpu.VMEM((num_types, half), jnp.float32),
            pltpu.VMEM_SHARED((NUM_BASIS, embed_dim), jnp.float32),
            pltpu.VMEM_SHARED((n_nodes, 8), jnp.float32),
            pltpu.SMEM((NUM_BASIS, embed_dim), jnp.float32),
            pltpu.VMEM((c,), jnp.int32),
            pltpu.VMEM((c,), jnp.int32),
            pltpu.VMEM((c, 8), jnp.float32),
            pltpu.VMEM((c, 8), jnp.float32),
            pltpu.VMEM((c, embed_dim), jnp.float32),
            pltpu.SemaphoreType.DMA,
            pltpu.SemaphoreType.DMA,
        ],
    )


@jax.jit
def kernel(pos, edge_index, atom_types, type_embeddings, W):
    n_nodes = pos.shape[0]
    n_edges = edge_index.shape[1]
    num_types, half = type_embeddings.shape[1], type_embeddings.shape[2]
    run = _make_run(n_nodes, n_edges, num_types, 2 * half, c=400)
    tval = atom_types.astype(jnp.float32)[:, None]
    ptab = jnp.concatenate(
        [pos, tval, jnp.zeros((n_nodes, 4), jnp.float32)], axis=1)
    ei_flat = edge_index.reshape(2 * n_edges)
    return run(ptab, ei_flat, type_embeddings, W)


# speedup vs baseline: 1.0880x; 1.0880x over previous
import functools

import jax
import jax.numpy as jnp
import numpy as np
from jax import lax
from jax.experimental import pallas as pl
from jax.experimental.pallas import tpu as pltpu
from jax.experimental.pallas import tpu_sc as plsc

NUM_BASIS = 8
R_MAX = 6.0
NC, NS, L = 2, 16, 16
NW = NC * NS


def _f(v):
    return jnp.float32(v)


def _make_run(n_nodes, n_edges, num_types, embed_dim, c, interpret=False):
    e_tile = n_edges // NW
    nchunk = e_tile // c
    assert e_tile * NW == n_edges and nchunk * c == e_tile and c % L == 0

    def _sc_body(pos_hbm, ei_hbm, te_hbm, w_hbm, out_hbm,
                 te0_v, te1_v, w_sh, pos_sh, w_s, src_v, dst_v, psrc_v,
                 pdst_v, out_v, sem1, sem2):
        cid = lax.axis_index("c")
        sid = lax.axis_index("s")
        wid = sid * NC + cid
        base0 = wid * e_tile

        pltpu.sync_copy(te_hbm.at[0], te0_v)
        pltpu.sync_copy(te_hbm.at[1], te1_v)

        @pl.when(sid == 0)
        def _stage_shared():
            pltpu.sync_copy(w_hbm, w_sh)
            pltpu.sync_copy(pos_hbm, pos_sh)

        plsc.subcore_barrier()
        pltpu.sync_copy(w_sh, w_s)

        iota = lax.iota(jnp.int32, L)
        col = [jnp.full((L,), j, jnp.int32) for j in range(4)]

        nsub = c // 80

        @pl.loop(0, nchunk)
        def _chunk(i):
            base = base0 + i * c
            di1 = pltpu.async_copy(ei_hbm.at[pl.ds(base, c)], src_v, sem1)
            di2 = pltpu.async_copy(ei_hbm.at[pl.ds(n_edges + base, c)],
                                   dst_v, sem1)
            di1.wait()
            di2.wait()
            descs = []
            for j in range(nsub):
                sl = pl.ds(j * 80, 80)
                descs.append(pltpu.async_copy(
                    pos_sh.at[src_v.at[sl]], psrc_v.at[sl, :], sem2))
                descs.append(pltpu.async_copy(
                    pos_sh.at[dst_v.at[sl]], pdst_v.at[sl, :], sem2))
            for d in descs:
                d.wait()

            for g in range(c // L):
                rows = iota + g * L
                x1 = plsc.load_gather(psrc_v, [rows, col[0]])
                y1 = plsc.load_gather(psrc_v, [rows, col[1]])
                z1 = plsc.load_gather(psrc_v, [rows, col[2]])
                x2 = plsc.load_gather(pdst_v, [rows, col[0]])
                y2 = plsc.load_gather(pdst_v, [rows, col[1]])
                z2 = plsc.load_gather(pdst_v, [rows, col[2]])
                tsrc = plsc.load_gather(psrc_v, [rows, col[3]]).astype(
                    jnp.int32)
                tdst = plsc.load_gather(pdst_v, [rows, col[3]]).astype(
                    jnp.int32)

                dx = x2 - x1
                dy = y2 - y1
                dz = z2 - z1
                d2 = dx * dx + dy * dy + dz * dz + _f(1e-12)

                ii = plsc.bitcast(d2, jnp.int32)
                ii = jnp.int32(0x5F3759DF) - lax.shift_right_logical(ii, 1)
                y = plsc.bitcast(ii, jnp.float32)
                for _ in range(4):
                    y = y * (_f(1.5) - _f(0.5) * d2 * y * y)
                x = d2 * y

                theta = jnp.minimum(x, _f(R_MAX)) * _f(np.pi / R_MAX)
                r = jnp.minimum(theta, _f(np.pi) - theta)
                p2 = r * r
                s = _f(1.0 / 362880.0)
                s = s * p2 + _f(-1.0 / 5040.0)
                s = s * p2 + _f(1.0 / 120.0)
                s = s * p2 + _f(-1.0 / 6.0)
                sin_t = r + r * p2 * s
                cpoly = _f(-1.0 / 3628800.0)
                cpoly = cpoly * p2 + _f(1.0 / 40320.0)
                cpoly = cpoly * p2 + _f(-1.0 / 720.0)
                cpoly = cpoly * p2 + _f(1.0 / 24.0)
                cpoly = cpoly * p2 + _f(-0.5)
                cos_r = _f(1.0) + p2 * cpoly
                twoc = jnp.where(theta < _f(np.pi / 2.0), cos_r + cos_r,
                                 -(cos_r + cos_r))

                sins = [sin_t, twoc * sin_t]
                for _ in range(NUM_BASIS - 2):
                    sins.append(twoc * sins[-1] - sins[-2])

                xs = x * _f(1.0 / R_MAX)
                xs2 = xs * xs
                xs4 = xs2 * xs2
                xs6 = xs4 * xs2
                xs7 = xs6 * xs
                xs8 = xs7 * xs
                cut = (_f(1.0) - _f(28.0) * xs6 + _f(48.0) * xs7
                       - _f(21.0) * xs8)
                cut = jnp.where(xs < _f(1.0), cut, _f(0.0))

                m_all = cut * (y * _f(2.0 / R_MAX))

                for j in range(embed_dim):
                    acc = sins[0] * w_s[0, j]
                    for k in range(1, NUM_BASIS):
                        acc = acc + sins[k] * w_s[k, j]
                    if j < embed_dim // 2:
                        tej = plsc.load_gather(
                            te0_v, [tsrc, jnp.full((L,), j, jnp.int32)])
                    else:
                        tej = plsc.load_gather(
                            te1_v,
                            [tdst, jnp.full((L,), j - embed_dim // 2,
                                            jnp.int32)])
                    o = acc * m_all * tej
                    plsc.store_scatter(
                        out_v, [rows, jnp.full((L,), j, jnp.int32)], o)

            pltpu.sync_copy(out_v, out_hbm.at[pl.ds(base, c), :])

    mesh = plsc.VectorSubcoreMesh(core_axis_name="c", subcore_axis_name="s",
                                  num_cores=NC, num_subcores=NS)
    half = embed_dim // 2
    return pl.kernel(
        _sc_body,
        out_type=jax.ShapeDtypeStruct((n_edges, embed_dim), jnp.float32),
        mesh=mesh,
        interpret=interpret,
        compiler_params=pltpu.CompilerParams(needs_layout_passes=False,
                                             use_tc_tiling_on_sc=False),
        scratch_types=[
            pltpu.VMEM((num_types, half), jnp.float32),
            pltpu.VMEM((num_types, half), jnp.float32),
            pltpu.VMEM_SHARED((NUM_BASIS, embed_dim), jnp.float32),
            pltpu.VMEM_SHARED((n_nodes, 8), jnp.float32),
            pltpu.SMEM((NUM_BASIS, embed_dim), jnp.float32),
            pltpu.VMEM((c,), jnp.int32),
            pltpu.VMEM((c,), jnp.int32),
            pltpu.VMEM((c, 8), jnp.float32),
            pltpu.VMEM((c, 8), jnp.float32),
            pltpu.VMEM((c, embed_dim), jnp.float32),
            pltpu.SemaphoreType.DMA,
            pltpu.SemaphoreType.DMA,
        ],
    )


@jax.jit
def kernel(pos, edge_index, atom_types, type_embeddings, W):
    n_nodes = pos.shape[0]
    n_edges = edge_index.shape[1]
    num_types, half = type_embeddings.shape[1], type_embeddings.shape[2]
    run = _make_run(n_nodes, n_edges, num_types, 2 * half, c=400)
    tval = atom_types.astype(jnp.float32)[:, None]
    ptab = jnp.concatenate(
        [pos, tval, jnp.zeros((n_nodes, 4), jnp.float32)], axis=1)
    ei_flat = edge_index.reshape(2 * n_edges)
    return run(ptab, ei_flat, type_embeddings, W)


# speedup vs baseline: 1.1010x; 1.0120x over previous
import functools

import jax
import jax.numpy as jnp
import numpy as np
from jax import lax
from jax.experimental import pallas as pl
from jax.experimental.pallas import tpu as pltpu
from jax.experimental.pallas import tpu_sc as plsc

NUM_BASIS = 8
R_MAX = 6.0
NC, NS, L = 2, 16, 16
NW = NC * NS


def _f(v):
    return jnp.float32(v)


def _make_run(n_nodes, n_edges, num_types, embed_dim, c):
    e_tile = n_edges // NW
    nchunk = e_tile // c
    assert e_tile * NW == n_edges and nchunk * c == e_tile
    assert c % 80 == 0 and nchunk >= 3 and embed_dim == 16
    nsub = c // 80
    half = embed_dim // 2

    def _sc_body(pos_hbm, ei_hbm, te_hbm, w_hbm, out_hbm,
                 te0_v, te1_v, w_sh, pos_sh, w_s, idx_v, prow_v, out_v,
                 sem_i0, sem_i1, sem_g0, sem_g1, sem_o0, sem_o1):
        sem_i = [sem_i0, sem_i1]
        sem_g = [sem_g0, sem_g1]
        sem_o = [sem_o0, sem_o1]
        cid = lax.axis_index("c")
        sid = lax.axis_index("s")
        wid = sid * NC + cid
        base0 = wid * e_tile

        pltpu.sync_copy(te_hbm.at[0], te0_v)
        pltpu.sync_copy(te_hbm.at[1], te1_v)

        @pl.when(sid == 0)
        def _stage_shared():
            pltpu.sync_copy(w_hbm, w_sh)
            pltpu.sync_copy(pos_hbm, pos_sh)

        plsc.subcore_barrier()
        pltpu.sync_copy(w_sh, w_s)

        iota = lax.iota(jnp.int32, L)
        col = [jnp.full((L,), j, jnp.int32) for j in range(4)]

        def idx_copies(k, b):
            base = base0 + k * c
            return (
                pltpu.make_async_copy(
                    ei_hbm.at[pl.ds(base, c)],
                    idx_v.at[pl.ds(b * (2 * c), c)], sem_i[b]),
                pltpu.make_async_copy(
                    ei_hbm.at[pl.ds(n_edges + base, c)],
                    idx_v.at[pl.ds(b * (2 * c) + c, c)], sem_i[b]),
            )

        def gather_copies(b):
            out = []
            for j in range(2 * nsub):
                sl = pl.ds(b * (2 * c) + j * 80, 80)
                out.append(pltpu.make_async_copy(
                    pos_sh.at[idx_v.at[sl]], prow_v.at[sl, :], sem_g[b]))
            return out

        def out_copy(k, b):
            base = base0 + k * c
            return pltpu.make_async_copy(
                out_v.at[pl.ds(b * c * embed_dim, c * embed_dim)],
                out_hbm.at[pl.ds(base * embed_dim, c * embed_dim)],
                sem_o[b])

        def compute(k, b):
            bc = b * (2 * c)
            for g in range(c // L):
                rows_s = iota + (bc + g * L)
                rows_d = iota + (bc + c + g * L)
                x1 = plsc.load_gather(prow_v, [rows_s, col[0]])
                y1 = plsc.load_gather(prow_v, [rows_s, col[1]])
                z1 = plsc.load_gather(prow_v, [rows_s, col[2]])
                x2 = plsc.load_gather(prow_v, [rows_d, col[0]])
                y2 = plsc.load_gather(prow_v, [rows_d, col[1]])
                z2 = plsc.load_gather(prow_v, [rows_d, col[2]])
                tsrc = plsc.load_gather(prow_v, [rows_s, col[3]]).astype(
                    jnp.int32)
                tdst = plsc.load_gather(prow_v, [rows_d, col[3]]).astype(
                    jnp.int32)

                dx = x2 - x1
                dy = y2 - y1
                dz = z2 - z1
                d2 = dx * dx + dy * dy + dz * dz + _f(1e-12)

                ii = plsc.bitcast(d2, jnp.int32)
                ii = jnp.int32(0x5F3759DF) - lax.shift_right_logical(ii, 1)
                y = plsc.bitcast(ii, jnp.float32)
                for _ in range(4):
                    y = y * (_f(1.5) - _f(0.5) * d2 * y * y)
                x = d2 * y

                theta = jnp.minimum(x, _f(R_MAX)) * _f(np.pi / R_MAX)
                r = jnp.minimum(theta, _f(np.pi) - theta)
                p2 = r * r
                s = _f(1.0 / 362880.0)
                s = s * p2 + _f(-1.0 / 5040.0)
                s = s * p2 + _f(1.0 / 120.0)
                s = s * p2 + _f(-1.0 / 6.0)
                sin_t = r + r * p2 * s
                cpoly = _f(-1.0 / 3628800.0)
                cpoly = cpoly * p2 + _f(1.0 / 40320.0)
                cpoly = cpoly * p2 + _f(-1.0 / 720.0)
                cpoly = cpoly * p2 + _f(1.0 / 24.0)
                cpoly = cpoly * p2 + _f(-0.5)
                cos_r = _f(1.0) + p2 * cpoly
                twoc = jnp.where(theta < _f(np.pi / 2.0), cos_r + cos_r,
                                 -(cos_r + cos_r))

                xs = x * _f(1.0 / R_MAX)
                xs2 = xs * xs
                xs4 = xs2 * xs2
                xs6 = xs4 * xs2
                xs7 = xs6 * xs
                xs8 = xs7 * xs
                cut = (_f(1.0) - _f(28.0) * xs6 + _f(48.0) * xs7
                       - _f(21.0) * xs8)
                cut = jnp.where(xs < _f(1.0), cut, _f(0.0))
                m_all = cut * (y * _f(2.0 / R_MAX))

                s1 = sin_t * m_all
                sins = [s1, twoc * s1]
                for _ in range(NUM_BASIS - 2):
                    sins.append(twoc * sins[-1] - sins[-2])

                ro = lax.shift_left(iota + (b * c + g * L), 4)
                for j in range(embed_dim):
                    acc = sins[0] * w_s[0, j]
                    for k2 in range(1, NUM_BASIS):
                        acc = acc + sins[k2] * w_s[k2, j]
                    if j < half:
                        tej = plsc.load_gather(
                            te0_v, [tsrc, jnp.full((L,), j, jnp.int32)])
                    else:
                        tej = plsc.load_gather(
                            te1_v, [tdst, jnp.full((L,), j - half,
                                                   jnp.int32)])
                    plsc.store_scatter(
                        out_v, [ro + jnp.int32(j)], acc * tej)

        for d in idx_copies(0, 0):
            d.start()
        for d in idx_copies(0, 0):
            d.wait()
        for d in gather_copies(0):
            d.start()
        for d in idx_copies(1, 1):
            d.start()

        def pre_block(i, b):
            nb = 1 - b

            @pl.when(i + 1 < nchunk)
            def _():
                for d in idx_copies(i + 1, nb):
                    d.wait()
                for d in gather_copies(nb):
                    d.start()

            for d in gather_copies(b):
                d.wait()

            @pl.when(i >= 2)
            def _():
                out_copy(i - 2, b).wait()

        def post_block(i, b):
            out_copy(i, b).start()

            @pl.when(i + 2 < nchunk)
            def _():
                for d in idx_copies(i + 2, b):
                    d.start()

        @pl.loop(0, nchunk)
        def _chunk(i):
            par = lax.rem(i, 2)

            @pl.when(par == 0)
            def _():
                pre_block(i, 0)

            @pl.when(par == 1)
            def _():
                pre_block(i, 1)

            compute(i, par)

            @pl.when(par == 0)
            def _():
                post_block(i, 0)

            @pl.when(par == 1)
            def _():
                post_block(i, 1)

        out_copy(nchunk - 2, (nchunk - 2) % 2).wait()
        out_copy(nchunk - 1, (nchunk - 1) % 2).wait()

    mesh = plsc.VectorSubcoreMesh(core_axis_name="c", subcore_axis_name="s",
                                  num_cores=NC, num_subcores=NS)
    return pl.kernel(
        _sc_body,
        out_type=jax.ShapeDtypeStruct((n_edges * embed_dim,), jnp.float32),
        mesh=mesh,
        compiler_params=pltpu.CompilerParams(needs_layout_passes=False,
                                             use_tc_tiling_on_sc=False),
        scratch_types=[
            pltpu.VMEM((num_types, half), jnp.float32),
            pltpu.VMEM((num_types, half), jnp.float32),
            pltpu.VMEM_SHARED((NUM_BASIS, embed_dim), jnp.float32),
            pltpu.VMEM_SHARED((n_nodes, 8), jnp.float32),
            pltpu.SMEM((NUM_BASIS, embed_dim), jnp.float32),
            pltpu.VMEM((4 * c,), jnp.int32),
            pltpu.VMEM((4 * c, 8), jnp.float32),
            pltpu.VMEM((2 * c * embed_dim,), jnp.float32),
            pltpu.SemaphoreType.DMA,
            pltpu.SemaphoreType.DMA,
            pltpu.SemaphoreType.DMA,
            pltpu.SemaphoreType.DMA,
            pltpu.SemaphoreType.DMA,
            pltpu.SemaphoreType.DMA,
        ],
    )


@jax.jit
def kernel(pos, edge_index, atom_types, type_embeddings, W):
    n_nodes = pos.shape[0]
    n_edges = edge_index.shape[1]
    num_types, half = type_embeddings.shape[1], type_embeddings.shape[2]
    run = _make_run(n_nodes, n_edges, num_types, 2 * half, c=400)
    tval = atom_types.astype(jnp.float32)[:, None]
    ptab = jnp.concatenate(
        [pos, tval, jnp.zeros((n_nodes, 4), jnp.float32)], axis=1)
    ei_flat = edge_index.reshape(2 * n_edges)
    out = run(ptab, ei_flat, type_embeddings, W)
    return out.reshape(n_edges, 2 * half)
